# speedup vs baseline: 8.8661x; 8.8661x over previous
import functools

import jax
import jax.numpy as jnp
from jax import lax
from jax.experimental import pallas as pl
from jax.experimental.pallas import tpu as pltpu

_K = 20


def _knn_body(pbf_ref, pbft_ref, p2r_ref, p2c_ref, idx_ref, *, n_cols, k):
    a = pbf_ref[...]
    bt = pbft_ref[...]
    dot = lax.dot_general(a, bt, (((1,), (0,)), ((), ())),
                          preferred_element_type=jnp.float32)
    d2 = (p2r_ref[...] + p2c_ref[...]) - 2.0 * dot
    work = -d2
    br = work.shape[0]
    iota = lax.broadcasted_iota(jnp.int32, (br, n_cols), 1)
    big = jnp.int32(2 ** 30)
    neginf = jnp.float32(-jnp.inf)
    cols = []
    for _ in range(k):
        mx = jnp.max(work, axis=1, keepdims=True)
        eq = work == mx
        ci = jnp.min(jnp.where(eq, iota, big), axis=1, keepdims=True)
        cols.append(ci)
        work = jnp.where(iota == ci, neginf, work)
    idx_ref[...] = jnp.concatenate(cols, axis=1)


def _knn_topk_idx(pts):
    n = pts.shape[1]
    p = pts[0]
    p2 = jnp.sum(pts * pts, axis=-1)[0]
    pbf = p.astype(jnp.bfloat16)
    pbft = pbf.T
    p2r = p2[:, None]
    p2c = p2[None, :]
    br = 200
    assert n % br == 0
    grid = (n // br,)
    idx = pl.pallas_call(
        functools.partial(_knn_body, n_cols=n, k=_K),
        grid=grid,
        in_specs=[
            pl.BlockSpec((br, 3), lambda i: (i, 0)),
            pl.BlockSpec((3, n), lambda i: (0, 0)),
            pl.BlockSpec((br, 1), lambda i: (i, 0)),
            pl.BlockSpec((1, n), lambda i: (0, 0)),
        ],
        out_specs=pl.BlockSpec((br, _K), lambda i: (i, 0)),
        out_shape=jax.ShapeDtypeStruct((n, _K), jnp.int32),
    )(pbf, pbft, p2r, p2c)
    return idx[None]


def _signs(normals, idx):
    _, n, _ = normals.shape
    nrm = normals[0]
    ix = idx[0]
    k = ix.shape[1]
    c = ix[:, 0]
    nc = nrm[c]
    nm = nrm[ix]
    dc0 = (nm[..., 0] * nc[:, None, 0] + nm[..., 1] * nc[:, None, 1]) \
        + nm[..., 2] * nc[:, None, 2]
    pos = dc0 > 0
    jrow = jnp.broadcast_to(jnp.arange(n, dtype=jnp.int32)[:, None], (n, k))

    def cond_fn(state):
        _, cur, _ = state
        return jnp.any(cur)

    def body_fn(state):
        s, cur, sched = state
        sc = s[c]
        val = jnp.where(pos, sc[:, None], -sc[:, None])
        wj = jnp.full(n, -1, jnp.int32).at[ix.reshape(-1)].max(
            jnp.where(cur[:, None], jrow, -1).reshape(-1))
        win = cur[:, None] & (jrow == wj[ix])
        tgt_eff = jnp.where(win, ix, n)
        s_new = s.at[tgt_eff.reshape(-1)].set(val.reshape(-1), mode="drop")
        sched_new = sched | cur
        touch_t = jnp.where(cur[:, None], ix[:, 1:], n)
        touched = jnp.zeros(n, bool).at[touch_t.reshape(-1)].set(
            True, mode="drop")
        cur_new = touched & ~sched_new
        return s_new, cur_new, sched_new

    s0 = jnp.ones(n, dtype=nrm.dtype)
    cur0 = jnp.zeros(n, bool).at[0].set(True)
    sched0 = jnp.zeros(n, bool)
    s, _, _ = lax.while_loop(cond_fn, body_fn, (s0, cur0, sched0))
    return s[None]


def kernel(pointscloud):
    idx = _knn_topk_idx(pointscloud)
    knn = jax.vmap(lambda a, i: a[i])(pointscloud, idx)
    centered = knn - knn.mean(axis=-2, keepdims=True)
    covs = jnp.matmul(jnp.swapaxes(centered, -1, -2), centered) / (
        centered.shape[-1] - 1)
    eigvals, eigvecs = jnp.linalg.eigh(covs)
    frames = jnp.swapaxes(eigvecs, -1, -2)
    normals = frames[:, :, 0, :]
    s = _signs(normals, idx)
    frames = frames.at[:, :, 0, :].set(normals * s[..., None])
    det = jnp.linalg.det(frames)
    frames = frames.at[:, :, 1, :].set(frames[:, :, 1, :] * det[..., None])
    return frames


# speedup vs baseline: 23.6180x; 2.6639x over previous
import functools

import jax
import jax.numpy as jnp
from jax import lax
from jax.experimental import pallas as pl
from jax.experimental.pallas import tpu as pltpu
from jax.experimental.pallas import tpu_sc as plsc

_K = 20
_N = 10000


def _knn_body(pbf_ref, pbft_ref, p2r_ref, p2c_ref, idx_ref, *, n_cols, k):
    a = pbf_ref[...]
    bt = pbft_ref[...]
    dot = lax.dot_general(a, bt, (((1,), (0,)), ((), ())),
                          preferred_element_type=jnp.float32)
    d2 = (p2r_ref[...] + p2c_ref[...]) - 2.0 * dot
    work = -d2
    br = work.shape[0]
    iota = lax.broadcasted_iota(jnp.int32, (br, n_cols), 1)
    big = jnp.int32(2 ** 30)
    neginf = jnp.float32(-jnp.inf)
    cols = []
    for _ in range(k):
        mx = jnp.max(work, axis=1, keepdims=True)
        eq = work == mx
        ci = jnp.min(jnp.where(eq, iota, big), axis=1, keepdims=True)
        cols.append(ci)
        work = jnp.where(iota == ci, neginf, work)
    idx_ref[...] = jnp.concatenate(cols, axis=1)


def _knn_topk_idx(pts):
    n = pts.shape[1]
    p = pts[0]
    p2 = jnp.sum(pts * pts, axis=-1)[0]
    pbf = p.astype(jnp.bfloat16)
    pbft = pbf.T
    p2r = p2[:, None]
    p2c = p2[None, :]
    br = 200
    assert n % br == 0
    grid = (n // br,)
    idx = pl.pallas_call(
        functools.partial(_knn_body, n_cols=n, k=_K),
        grid=grid,
        in_specs=[
            pl.BlockSpec((br, 3), lambda i: (i, 0)),
            pl.BlockSpec((3, n), lambda i: (0, 0)),
            pl.BlockSpec((br, 1), lambda i: (i, 0)),
            pl.BlockSpec((1, n), lambda i: (0, 0)),
        ],
        out_specs=pl.BlockSpec((br, _K), lambda i: (i, 0)),
        out_shape=jax.ShapeDtypeStruct((n, _K), jnp.int32),
    )(pbf, pbft, p2r, p2c)
    return idx[None]


_RPT = 640
_ACH = 80
_ROWW = 16


def _sc_body(nx_h, ny_h, nz_h, idx_h, out_h,
             nx_v, ny_v, nz_v, idx_v, buf_v,
             s_v, snap_v, touch_v, sched_v, cur_v, row_v, table_sp):
    c_ax = lax.axis_index("c")
    s_ax = lax.axis_index("s")
    iota = lax.broadcasted_iota(jnp.int32, (16,), 0)
    zeros16 = jnp.zeros((16,), jnp.int32)

    @pl.when(c_ax == 0)
    def _phase_a():
        pltpu.sync_copy(nx_h, nx_v)
        pltpu.sync_copy(ny_h, ny_v)
        pltpu.sync_copy(nz_h, nz_v)
        base_row = s_ax * _RPT

        def chunk_body(ch, carry):
            row0 = base_row + ch * _ACH
            pltpu.sync_copy(idx_h.at[pl.ds(row0, _ACH)], idx_v)

            def row_body(jj, carry2):
                m0 = idx_v[jj, 0:16]
                m1 = idx_v[jj, 4:20]
                cvec = plsc.load_gather(
                    idx_v, [jnp.full((16,), jj, jnp.int32), zeros16])
                cx = plsc.load_gather(nx_v, [cvec])
                cy = plsc.load_gather(ny_v, [cvec])
                cz = plsc.load_gather(nz_v, [cvec])

                def pos_of(m):
                    mx = plsc.load_gather(nx_v, [m])
                    my = plsc.load_gather(ny_v, [m])
                    mz = plsc.load_gather(nz_v, [m])
                    dc = (mx * cx + my * cy) + mz * cz
                    return jnp.where(dc > 0.0, jnp.int32(1), jnp.int32(0))

                e0 = m0 | (pos_of(m0) << 14)
                e1 = m1 | (pos_of(m1) << 14)
                buf_v[pl.ds(jj * _ROWW, 16)] = e0 | (e1 << 16)
                return carry2

            lax.fori_loop(0, _ACH, row_body, 0)
            pltpu.sync_copy(buf_v,
                            table_sp.at[pl.ds(row0 * _ROWW, _ACH * _ROWW)])
            return carry

        nch = jnp.where(s_ax == 15, (_N - 15 * _RPT) // _ACH, _RPT // _ACH)
        lax.fori_loop(0, nch, chunk_body, 0)

    plsc.subcore_barrier()

    @pl.when((c_ax == 0) & (s_ax == 0))
    def _phase_b():
        ones16f = jnp.ones((16,), jnp.float32)
        neg16 = jnp.full((16,), -1, jnp.int32)

        def init_body(i, carry):
            s_v[pl.ds(i * 16, 16)] = ones16f
            touch_v[pl.ds(i * 16, 16)] = neg16
            sched_v[pl.ds(i * 16, 16)] = zeros16
            return carry

        lax.fori_loop(0, _N // 16, init_body, 0)
        touch_v[0:16] = jnp.where(iota == 0, jnp.int32(0), jnp.int32(-1))

        def wave_cond(carry):
            return carry[1] > 0

        def wave_body(carry):
            wave, _ = carry
            wv = jnp.full((16,), wave, jnp.int32)

            def p1(i, cnt):
                sl = pl.ds(i * 16, 16)
                snap_v[sl] = s_v[sl]
                t = touch_v[sl]
                sc_f = sched_v[sl]
                cm = jnp.where((t == wave - 1) & (sc_f == 0),
                               jnp.int32(1), jnp.int32(0))
                cur_v[sl] = cm
                sched_v[sl] = jnp.where(cm == 1, jnp.int32(1), sc_f)
                return cnt + jnp.sum(cm)

            nproc = lax.fori_loop(0, _N // 16, p1, jnp.int32(0))

            def p2(i, c2):
                cm = cur_v[pl.ds(i * 16, 16)]

                @pl.when(jnp.max(cm) == 1)
                def _blk():
                    for t in range(16):
                        @pl.when(cm[t] == 1)
                        def _center():
                            j = i * 16 + t
                            pltpu.sync_copy(
                                table_sp.at[pl.ds(j * _ROWW, _ROWW)], row_v)
                            w = row_v[0:16]
                            e0 = w & 0xFFFF
                            e1 = (w >> 16) & 0xFFFF
                            tg0 = e0 & 0x3FFF
                            ps0 = e0 >> 14
                            tg1 = e1 & 0x3FFF
                            ps1 = e1 >> 14
                            cvec = jnp.full((16,), tg0[0], jnp.int32)
                            scv = plsc.load_gather(snap_v, [cvec])
                            nscv = -scv
                            val0 = jnp.where(ps0 == 1, scv, nscv)
                            val1 = jnp.where(ps1 == 1, scv, nscv)
                            plsc.store_scatter(s_v, [tg0], val0)
                            plsc.store_scatter(s_v, [tg1], val1)
                            t0 = jnp.where(iota == 0, jnp.int32(_N), tg0)
                            plsc.store_scatter(touch_v, [t0], wv)
                            plsc.store_scatter(touch_v, [tg1], wv)
                return c2

            lax.fori_loop(0, _N // 16, p2, 0)
            return (wave + jnp.int32(1), nproc)

        lax.while_loop(wave_cond, wave_body, (jnp.int32(1), jnp.int32(1)))
        pltpu.sync_copy(s_v, out_h)


def _signs_sc(normals, idx):
    nt = jnp.transpose(normals[0])
    mesh = plsc.VectorSubcoreMesh(core_axis_name="c", subcore_axis_name="s")
    fn = pl.kernel(
        _sc_body,
        mesh=mesh,
        compiler_params=pltpu.CompilerParams(needs_layout_passes=False),
        out_type=jax.ShapeDtypeStruct((_N,), jnp.float32),
        scratch_types=[
            pltpu.VMEM((_N,), jnp.float32),
            pltpu.VMEM((_N,), jnp.float32),
            pltpu.VMEM((_N,), jnp.float32),
            pltpu.VMEM((_ACH, _K), jnp.int32),
            pltpu.VMEM((_ACH * _ROWW,), jnp.int32),
            pltpu.VMEM((_N,), jnp.float32),
            pltpu.VMEM((_N,), jnp.float32),
            pltpu.VMEM((_N + 16,), jnp.int32),
            pltpu.VMEM((_N,), jnp.int32),
            pltpu.VMEM((_N,), jnp.int32),
            pltpu.VMEM((_ROWW,), jnp.int32),
            pltpu.VMEM_SHARED((_N * _ROWW,), jnp.int32),
        ],
    )
    s = fn(nt[0], nt[1], nt[2], idx[0])
    return s[None]


def _signs(normals, idx):
    _, n, _ = normals.shape
    nrm = normals[0]
    ix = idx[0]
    k = ix.shape[1]
    c = ix[:, 0]
    nc = nrm[c]
    nm = nrm[ix]
    dc0 = (nm[..., 0] * nc[:, None, 0] + nm[..., 1] * nc[:, None, 1]) \
        + nm[..., 2] * nc[:, None, 2]
    pos = dc0 > 0
    jrow = jnp.broadcast_to(jnp.arange(n, dtype=jnp.int32)[:, None], (n, k))

    def cond_fn(state):
        _, cur, _ = state
        return jnp.any(cur)

    def body_fn(state):
        s, cur, sched = state
        sc = s[c]
        val = jnp.where(pos, sc[:, None], -sc[:, None])
        wj = jnp.full(n, -1, jnp.int32).at[ix.reshape(-1)].max(
            jnp.where(cur[:, None], jrow, -1).reshape(-1))
        win = cur[:, None] & (jrow == wj[ix])
        tgt_eff = jnp.where(win, ix, n)
        s_new = s.at[tgt_eff.reshape(-1)].set(val.reshape(-1), mode="drop")
        sched_new = sched | cur
        touch_t = jnp.where(cur[:, None], ix[:, 1:], n)
        touched = jnp.zeros(n, bool).at[touch_t.reshape(-1)].set(
            True, mode="drop")
        cur_new = touched & ~sched_new
        return s_new, cur_new, sched_new

    s0 = jnp.ones(n, dtype=nrm.dtype)
    cur0 = jnp.zeros(n, bool).at[0].set(True)
    sched0 = jnp.zeros(n, bool)
    s, _, _ = lax.while_loop(cond_fn, body_fn, (s0, cur0, sched0))
    return s[None]


def kernel(pointscloud):
    idx = _knn_topk_idx(pointscloud)
    knn = jax.vmap(lambda a, i: a[i])(pointscloud, idx)
    centered = knn - knn.mean(axis=-2, keepdims=True)
    covs = jnp.matmul(jnp.swapaxes(centered, -1, -2), centered) / (
        centered.shape[-1] - 1)
    eigvals, eigvecs = jnp.linalg.eigh(covs)
    frames = jnp.swapaxes(eigvecs, -1, -2)
    normals = frames[:, :, 0, :]
    s = _signs_sc(normals, idx)
    frames = frames.at[:, :, 0, :].set(normals * s[..., None])
    det = jnp.linalg.det(frames)
    frames = frames.at[:, :, 1, :].set(frames[:, :, 1, :] * det[..., None])
    return frames
